# baseline (speedup 1.0000x reference)
import jax
import jax.numpy as jnp
import numpy as np
from jax.experimental import pallas as pl
from jax.experimental.pallas import tpu as pltpu

B, T, C = 2, 2048, 768
E, MIN_E, HD = 16, 2, 64
BLK = 256


def _fold_sum(x):
    while x.shape[1] > HD:
        h = x.shape[1] // 2
        x = x[:, :h] + x[:, h:]
    return x


def _body(x_ref, sim_ref, gates_ref, w_all_ref, ost_ref, out_ref,
          kscr, vscr):
    qi = pl.program_id(1)

    @pl.when(qi == 0)
    def _init():
        kscr[...] = jnp.zeros_like(kscr)
        vscr[...] = jnp.zeros_like(vscr)

    x = x_ref[...]
    xnorm = jnp.sqrt(jnp.sum(x * x, axis=1, keepdims=True))
    hn = x / jnp.maximum(xnorm, 1e-12)
    sim = sim_ref[...]
    snorm = jnp.sqrt(jnp.sum(sim * sim, axis=0, keepdims=True))
    sn = sim / jnp.maximum(snorm, 1e-12)
    logits = jnp.dot(hn, sn, preferred_element_type=jnp.float32)
    logits = logits - jax.nn.sigmoid(gates_ref[...])
    gated = jnp.maximum(logits, 0.0)
    mask = (gated > 0.0).astype(jnp.float32)
    inactive = jnp.sum(mask, axis=1, keepdims=True) == 0.0
    iota = jax.lax.broadcasted_iota(jnp.int32, logits.shape, 1)
    max1 = jnp.max(logits, axis=1, keepdims=True)
    idx1 = jnp.min(jnp.where(logits == max1, iota, E), axis=1, keepdims=True)
    l2 = jnp.where(iota == idx1, -jnp.inf, logits)
    max2 = jnp.max(l2, axis=1, keepdims=True)
    idx2 = jnp.min(jnp.where(l2 == max2, iota, E), axis=1, keepdims=True)
    fb = jnp.logical_or(iota == idx1, iota == idx2)
    mask = jnp.where(jnp.logical_and(inactive, fb), 1.0, mask)
    gm = jnp.where(mask > 0.0, gated, jnp.float32(-1e9))
    gm_max = jnp.max(gm, axis=1, keepdims=True)
    pw = jnp.exp(gm - gm_max)
    w = (pw / jnp.sum(pw, axis=1, keepdims=True)) * mask
    P = jnp.dot(x.astype(jnp.bfloat16), w_all_ref[...],
                preferred_element_type=jnp.float32)
    mexp = jnp.concatenate(
        [jnp.broadcast_to(mask[:, e:e + 1], (BLK, HD)) for e in range(E)],
        axis=1)
    q = _fold_sum(P[:, :E * HD] * mexp)
    k = _fold_sum(P[:, E * HD:2 * E * HD] * mexp)
    v = _fold_sum(P[:, 2 * E * HD:] * mexp)
    scale2 = jnp.float32(np.log2(np.e) / np.sqrt(HD))
    qb = q.astype(jnp.bfloat16)
    kb = k.astype(jnp.bfloat16)
    vb = v.astype(jnp.bfloat16)
    s = jax.lax.dot_general(qb, kscr[...], (((1,), (1,)), ((), ())),
                            preferred_element_type=jnp.float32) * scale2
    sd = jax.lax.dot_general(qb, kb, (((1,), (1,)), ((), ())),
                             preferred_element_type=jnp.float32) * scale2
    rl = jax.lax.broadcasted_iota(jnp.int32, (BLK, BLK), 0)
    cl = jax.lax.broadcasted_iota(jnp.int32, (BLK, BLK), 1)
    sd = jnp.where(cl <= rl, sd, jnp.float32(-1e9))
    p = jnp.exp2(s)
    pd = jnp.exp2(sd)
    o = (jnp.dot(p.astype(jnp.bfloat16), vscr[...],
                 preferred_element_type=jnp.float32)
         + jnp.dot(pd.astype(jnp.bfloat16), vb,
                   preferred_element_type=jnp.float32))
    denom = (jnp.sum(p, axis=1, keepdims=True)
             - jnp.float32(T) + jnp.float32(BLK) * qi
             + jnp.sum(pd, axis=1, keepdims=True))
    o = o / denom
    kscr[pl.ds(qi * BLK, BLK), :] = kb
    vscr[pl.ds(qi * BLK, BLK), :] = vb
    a = jnp.concatenate([w[:, e:e + 1] * o for e in range(E)], axis=1)
    out_ref[...] = jnp.dot(a.astype(jnp.bfloat16), ost_ref[...],
                           preferred_element_type=jnp.float32)


def kernel(hidden_states, sim_matrix, gates, q_proj, k_proj, v_proj, o_proj):
    flat = hidden_states.reshape(B * T, C)
    w_all = jnp.concatenate(
        [q_proj.transpose(1, 0, 2).reshape(C, E * HD),
         k_proj.transpose(1, 0, 2).reshape(C, E * HD),
         v_proj.transpose(1, 0, 2).reshape(C, E * HD)],
        axis=1).astype(jnp.bfloat16)
    gates2 = gates.reshape(1, E)
    o_st = o_proj.reshape(E * HD, C).astype(jnp.bfloat16)
    nq = T // BLK

    out = pl.pallas_call(
        _body,
        grid=(B, nq),
        in_specs=[
            pl.BlockSpec((BLK, C), lambda b, i: (b * nq + i, 0)),
            pl.BlockSpec((C, E), lambda b, i: (0, 0)),
            pl.BlockSpec((1, E), lambda b, i: (0, 0)),
            pl.BlockSpec((C, 3 * E * HD), lambda b, i: (0, 0)),
            pl.BlockSpec((E * HD, C), lambda b, i: (0, 0)),
        ],
        out_specs=pl.BlockSpec((BLK, C), lambda b, i: (b * nq + i, 0)),
        out_shape=jax.ShapeDtypeStruct((B * T, C), jnp.float32),
        scratch_shapes=[
            pltpu.VMEM((T, HD), jnp.bfloat16),
            pltpu.VMEM((T, HD), jnp.bfloat16),
        ],
    )(flat, sim_matrix, gates2, w_all, o_st)

    return out.reshape(B, T, C)


# speedup vs baseline: 1.0654x; 1.0654x over previous
import jax
import jax.numpy as jnp
import numpy as np
from jax.experimental import pallas as pl
from jax.experimental.pallas import tpu as pltpu

B, T, C = 2, 2048, 768
E, MIN_E, HD = 16, 2, 64
BLK = 256


def _fold_sum(x):
    while x.shape[1] > HD:
        h = x.shape[1] // 2
        x = x[:, :h] + x[:, h:]
    return x


def _body(x_ref, sim_ref, gates_ref, w_all_ref, ost_ref, out_ref,
          kscr, vscr):
    qi = pl.program_id(1)

    @pl.when(jnp.logical_and(pl.program_id(0) == 0, qi == 0))
    def _init():
        kscr[...] = jnp.zeros_like(kscr)
        vscr[...] = jnp.zeros_like(vscr)

    x = x_ref[...]
    xnorm = jnp.sqrt(jnp.sum(x * x, axis=1, keepdims=True))
    hn = x / jnp.maximum(xnorm, 1e-12)
    sim = sim_ref[...]
    snorm = jnp.sqrt(jnp.sum(sim * sim, axis=0, keepdims=True))
    sn = sim / jnp.maximum(snorm, 1e-12)
    logits = jnp.dot(hn, sn, preferred_element_type=jnp.float32)
    logits = logits - jax.nn.sigmoid(gates_ref[...])
    gated = jnp.maximum(logits, 0.0)
    mask = (gated > 0.0).astype(jnp.float32)
    inactive = jnp.sum(mask, axis=1, keepdims=True) == 0.0
    iota = jax.lax.broadcasted_iota(jnp.int32, logits.shape, 1)
    max1 = jnp.max(logits, axis=1, keepdims=True)
    idx1 = jnp.min(jnp.where(logits == max1, iota, E), axis=1, keepdims=True)
    l2 = jnp.where(iota == idx1, -jnp.inf, logits)
    max2 = jnp.max(l2, axis=1, keepdims=True)
    idx2 = jnp.min(jnp.where(l2 == max2, iota, E), axis=1, keepdims=True)
    fb = jnp.logical_or(iota == idx1, iota == idx2)
    mask = jnp.where(jnp.logical_and(inactive, fb), 1.0, mask)
    gm = jnp.where(mask > 0.0, gated, jnp.float32(-1e9))
    gm_max = jnp.max(gm, axis=1, keepdims=True)
    pw = jnp.exp(gm - gm_max)
    w = (pw / jnp.sum(pw, axis=1, keepdims=True)) * mask
    P = jnp.dot(x.astype(jnp.bfloat16), w_all_ref[...],
                preferred_element_type=jnp.float32)
    mexp = jnp.concatenate(
        [jnp.broadcast_to(mask[:, e:e + 1], (BLK, HD)) for e in range(E)],
        axis=1)
    q = _fold_sum(P[:, :E * HD] * mexp)
    k = _fold_sum(P[:, E * HD:2 * E * HD] * mexp)
    v = _fold_sum(P[:, 2 * E * HD:] * mexp)
    kscr[pl.ds(qi * BLK, BLK), :] = k.astype(jnp.bfloat16)
    vscr[pl.ds(qi * BLK, BLK), :] = v.astype(jnp.bfloat16)
    scale2 = jnp.float32(np.log2(np.e) / np.sqrt(HD))
    qb = q.astype(jnp.bfloat16)
    kb = k.astype(jnp.bfloat16)
    vb = v.astype(jnp.bfloat16)
    s = jax.lax.dot_general(qb, kscr[...], (((1,), (1,)), ((), ())),
                            preferred_element_type=jnp.float32) * scale2
    sd = jax.lax.dot_general(qb, kb, (((1,), (1,)), ((), ())),
                             preferred_element_type=jnp.float32) * scale2
    rl = jax.lax.broadcasted_iota(jnp.int32, (BLK, BLK), 0)
    cl = jax.lax.broadcasted_iota(jnp.int32, (BLK, BLK), 1)
    sd = jnp.where(cl <= rl, sd, jnp.float32(-1e9))
    valid = (jax.lax.broadcasted_iota(jnp.int32, (1, T), 1)
             < qi * BLK).astype(jnp.float32)
    p = jnp.exp2(s) * valid
    pd = jnp.exp2(sd)
    o = (jnp.dot(p.astype(jnp.bfloat16), vscr[...],
                 preferred_element_type=jnp.float32)
         + jnp.dot(pd.astype(jnp.bfloat16), vb,
                   preferred_element_type=jnp.float32))
    o = o / (jnp.sum(p, axis=1, keepdims=True)
             + jnp.sum(pd, axis=1, keepdims=True))
    a = jnp.concatenate([w[:, e:e + 1] * o for e in range(E)], axis=1)
    out_ref[...] = jnp.dot(a.astype(jnp.bfloat16), ost_ref[...],
                           preferred_element_type=jnp.float32)


def kernel(hidden_states, sim_matrix, gates, q_proj, k_proj, v_proj, o_proj):
    flat = hidden_states.reshape(B * T, C)
    w_all = jnp.concatenate(
        [q_proj.transpose(1, 0, 2).reshape(C, E * HD),
         k_proj.transpose(1, 0, 2).reshape(C, E * HD),
         v_proj.transpose(1, 0, 2).reshape(C, E * HD)],
        axis=1).astype(jnp.bfloat16)
    gates2 = gates.reshape(1, E)
    o_st = o_proj.reshape(E * HD, C).astype(jnp.bfloat16)
    nq = T // BLK

    out = pl.pallas_call(
        _body,
        grid=(B, nq),
        in_specs=[
            pl.BlockSpec((BLK, C), lambda b, i: (b * nq + i, 0)),
            pl.BlockSpec((C, E), lambda b, i: (0, 0)),
            pl.BlockSpec((1, E), lambda b, i: (0, 0)),
            pl.BlockSpec((C, 3 * E * HD), lambda b, i: (0, 0)),
            pl.BlockSpec((E * HD, C), lambda b, i: (0, 0)),
        ],
        out_specs=pl.BlockSpec((BLK, C), lambda b, i: (b * nq + i, 0)),
        out_shape=jax.ShapeDtypeStruct((B * T, C), jnp.float32),
        scratch_shapes=[
            pltpu.VMEM((T, HD), jnp.bfloat16),
            pltpu.VMEM((T, HD), jnp.bfloat16),
        ],
    )(flat, sim_matrix, gates2, w_all, o_st)

    return out.reshape(B, T, C)


# speedup vs baseline: 1.0688x; 1.0032x over previous
import jax
import jax.numpy as jnp
import numpy as np
from jax.experimental import pallas as pl
from jax.experimental.pallas import tpu as pltpu

B, T, C = 2, 2048, 768
E, MIN_E, HD = 16, 2, 64
BLK = 256


def _fold_sum(x):
    while x.shape[1] > HD:
        h = x.shape[1] // 2
        x = x[:, :h] + x[:, h:]
    return x


def _body(x_ref, sim_ref, gates_ref, w_all_ref, ost_ref, out_ref,
          kscr, vscr):
    qi = pl.program_id(1)

    @pl.when(jnp.logical_and(pl.program_id(0) == 0, qi == 0))
    def _init():
        kscr[...] = jnp.zeros_like(kscr)
        vscr[...] = jnp.zeros_like(vscr)

    x = x_ref[...]
    xnorm = jnp.sqrt(jnp.sum(x * x, axis=1, keepdims=True))
    sim = sim_ref[...]
    snorm = jnp.sqrt(jnp.sum(sim * sim, axis=0, keepdims=True))
    sn = sim / jnp.maximum(snorm, 1e-12)
    logits = jnp.dot(x, sn, preferred_element_type=jnp.float32)
    logits = (logits / jnp.maximum(xnorm, 1e-12)
              - jax.nn.sigmoid(gates_ref[...]))
    gated = jnp.maximum(logits, 0.0)
    mask = (gated > 0.0).astype(jnp.float32)
    inactive = jnp.sum(mask, axis=1, keepdims=True) == 0.0
    iota = jax.lax.broadcasted_iota(jnp.int32, logits.shape, 1)
    max1 = jnp.max(logits, axis=1, keepdims=True)
    idx1 = jnp.min(jnp.where(logits == max1, iota, E), axis=1, keepdims=True)
    l2 = jnp.where(iota == idx1, -jnp.inf, logits)
    max2 = jnp.max(l2, axis=1, keepdims=True)
    idx2 = jnp.min(jnp.where(l2 == max2, iota, E), axis=1, keepdims=True)
    fb = jnp.logical_or(iota == idx1, iota == idx2)
    mask = jnp.where(jnp.logical_and(inactive, fb), 1.0, mask)
    gm = jnp.where(mask > 0.0, gated, jnp.float32(-1e9))
    gm_max = jnp.max(gm, axis=1, keepdims=True)
    pw = jnp.exp(gm - gm_max)
    w = (pw / jnp.sum(pw, axis=1, keepdims=True)) * mask
    P = jnp.dot(x.astype(jnp.bfloat16), w_all_ref[...],
                preferred_element_type=jnp.float32)
    mexp = jnp.concatenate(
        [jnp.broadcast_to(mask[:, e:e + 1], (BLK, HD)) for e in range(E)],
        axis=1)
    q = _fold_sum(P[:, :E * HD] * mexp)
    k = _fold_sum(P[:, E * HD:2 * E * HD] * mexp)
    v = _fold_sum(P[:, 2 * E * HD:] * mexp)
    kscr[pl.ds(qi * BLK, BLK), :] = k.astype(jnp.bfloat16)
    vscr[pl.ds(qi * BLK, BLK), :] = v.astype(jnp.bfloat16)
    scale2 = jnp.float32(np.log2(np.e) / np.sqrt(HD))
    qb = q.astype(jnp.bfloat16)
    kb = k.astype(jnp.bfloat16)
    vb = v.astype(jnp.bfloat16)
    s = jax.lax.dot_general(qb, kscr[...], (((1,), (1,)), ((), ())),
                            preferred_element_type=jnp.float32) * scale2
    sd = jax.lax.dot_general(qb, kb, (((1,), (1,)), ((), ())),
                             preferred_element_type=jnp.float32) * scale2
    rl = jax.lax.broadcasted_iota(jnp.int32, (BLK, BLK), 0)
    cl = jax.lax.broadcasted_iota(jnp.int32, (BLK, BLK), 1)
    sd = jnp.where(cl <= rl, sd, jnp.float32(-1e9))
    valid = (jax.lax.broadcasted_iota(jnp.int32, (1, T), 1)
             < qi * BLK).astype(jnp.float32)
    p = jnp.exp2(s) * valid
    pd = jnp.exp2(sd)
    o = (jnp.dot(p.astype(jnp.bfloat16), vscr[...],
                 preferred_element_type=jnp.float32)
         + jnp.dot(pd.astype(jnp.bfloat16), vb,
                   preferred_element_type=jnp.float32))
    o = o / (jnp.sum(p, axis=1, keepdims=True)
             + jnp.sum(pd, axis=1, keepdims=True))
    a = jnp.concatenate([w[:, e:e + 1] * o for e in range(E)], axis=1)
    out_ref[...] = jnp.dot(a.astype(jnp.bfloat16), ost_ref[...],
                           preferred_element_type=jnp.float32)


def kernel(hidden_states, sim_matrix, gates, q_proj, k_proj, v_proj, o_proj):
    flat = hidden_states.reshape(B * T, C)
    w_all = jnp.concatenate(
        [q_proj.transpose(1, 0, 2).reshape(C, E * HD),
         k_proj.transpose(1, 0, 2).reshape(C, E * HD),
         v_proj.transpose(1, 0, 2).reshape(C, E * HD)],
        axis=1).astype(jnp.bfloat16)
    gates2 = gates.reshape(1, E)
    o_st = o_proj.reshape(E * HD, C).astype(jnp.bfloat16)
    nq = T // BLK

    out = pl.pallas_call(
        _body,
        grid=(B, nq),
        in_specs=[
            pl.BlockSpec((BLK, C), lambda b, i: (b * nq + i, 0)),
            pl.BlockSpec((C, E), lambda b, i: (0, 0)),
            pl.BlockSpec((1, E), lambda b, i: (0, 0)),
            pl.BlockSpec((C, 3 * E * HD), lambda b, i: (0, 0)),
            pl.BlockSpec((E * HD, C), lambda b, i: (0, 0)),
        ],
        out_specs=pl.BlockSpec((BLK, C), lambda b, i: (b * nq + i, 0)),
        out_shape=jax.ShapeDtypeStruct((B * T, C), jnp.float32),
        scratch_shapes=[
            pltpu.VMEM((T, HD), jnp.bfloat16),
            pltpu.VMEM((T, HD), jnp.bfloat16),
        ],
    )(flat, sim_matrix, gates2, w_all, o_st)

    return out.reshape(B, T, C)
